# baseline (speedup 1.0000x reference)
import functools

import jax
import jax.numpy as jnp
from jax import lax
from jax.experimental import pallas as pl
from jax.experimental.pallas import tpu as pltpu
from jax.experimental.pallas import tpu_sc as plsc

_DIST_MEAN = 9.578281194509781
_DIST_STD = 3.9656010701306283

_B = 16384
_NC = 2
_NS = 16
_NW = _NC * _NS
_BPW = _B // _NW

_DW = 16
_WW = 3
_TW = 8
_OUTW = _DW + _WW + _TW + 1
_WPAD = 4

_WAVE = 16
_NWAVES = _BPW // _WAVE


def _attr_body(drv_t, didx, week_tab, widx, time_tab, tidx, dist,
               out_t,
               didx_v, buf0, buf1, panel_v, week_v, time_v, widx_v, tidx_v,
               dist_v, sem0, sem1):
    wid = lax.axis_index("s") * _NC + lax.axis_index("c")
    base = wid * _BPW

    pltpu.sync_copy(didx.at[pl.ds(base, _BPW)], didx_v)
    pltpu.sync_copy(week_tab, week_v)
    pltpu.sync_copy(time_tab, time_v)
    pltpu.sync_copy(widx.at[pl.ds(base, _BPW)], widx_v)
    pltpu.sync_copy(tidx.at[pl.ds(base, _BPW)], tidx_v)
    pltpu.sync_copy(dist.at[pl.ds(base, _BPW)], dist_v)

    lanes = lax.iota(jnp.int32, 16)

    def fire(w, buf, sem):
        jv = didx_v[pl.ds(w * _WAVE, _WAVE)]
        for k in range(_WAVE):
            off = pl.multiple_of(jv[k] & -128, 128)
            pltpu.async_copy(drv_t.at[:, pl.ds(off, 128)], buf.at[k], sem)
        return jv

    def drain(buf, sem):
        for k in range(_WAVE):
            pltpu.make_async_copy(
                drv_t.at[:, pl.ds(0, 128)], buf.at[k], sem).wait()

    def extract(w, jv, buf):
        cols = jv & 127
        for k in range(_WAVE):
            vals = plsc.load_gather(
                buf, [jnp.full((16,), k, jnp.int32), lanes,
                      jnp.full((16,), cols[k], jnp.int32)])
            plsc.store_scatter(
                panel_v, [lanes, jnp.full((16,), w * _WAVE + k, jnp.int32)],
                vals)

    jv0 = fire(0, buf0, sem0)

    def fire_guarded(w):
        ws = jnp.where(w < _NWAVES, w, _NWAVES - 1)
        return fire(ws, buf0, sem0)

    def wave_pair(t, jv_carry):
        jv_a = jv_carry
        jv_b = fire(2 * t + 1, buf1, sem1)
        drain(buf0, sem0)
        extract(2 * t, jv_a, buf0)
        jv_next = fire_guarded(2 * t + 2)
        drain(buf1, sem1)
        extract(2 * t + 1, jv_b, buf1)
        return jv_next

    lax.fori_loop(0, _NWAVES // 2, wave_pair, jv0)
    drain(buf0, sem0)

    def fill(j, carry):
        r0 = j * 16
        wv = widx_v[pl.ds(r0, 16)]
        tv = tidx_v[pl.ds(r0, 16)]
        dv = dist_v[pl.ds(r0, 16)]
        wb = wv * _WPAD
        tb = tv * _TW
        rvec = r0 + lanes
        for c in range(_WW):
            vals = plsc.load_gather(week_v, [wb + c])
            plsc.store_scatter(
                panel_v, [jnp.full((16,), _DW + c, jnp.int32), rvec], vals)
        for c in range(_TW):
            vals = plsc.load_gather(time_v, [tb + c])
            plsc.store_scatter(
                panel_v, [jnp.full((16,), _DW + _WW + c, jnp.int32), rvec],
                vals)
        d = ((dv - _DIST_MEAN) / _DIST_STD - _DIST_MEAN) / _DIST_STD
        plsc.store_scatter(
            panel_v, [jnp.full((16,), _OUTW - 1, jnp.int32), rvec], d)
        return carry

    lax.fori_loop(0, _BPW // 16, fill, 0)

    pltpu.sync_copy(panel_v, out_t.at[:, pl.ds(base, _BPW)])


@jax.jit
def _attr_call(drv_t, didx, week_tab, widx, time_tab, tidx, dist):
    run = functools.partial(
        pl.kernel,
        out_type=jax.ShapeDtypeStruct((_OUTW, _B), jnp.float32),
        mesh=plsc.VectorSubcoreMesh(core_axis_name="c", subcore_axis_name="s"),
        scratch_types=[
            pltpu.VMEM((_BPW,), jnp.int32),
            pltpu.VMEM((_WAVE, _DW, 128), jnp.float32),
            pltpu.VMEM((_WAVE, _DW, 128), jnp.float32),
            pltpu.VMEM((_OUTW, _BPW), jnp.float32),
            pltpu.VMEM((32 * _WPAD,), jnp.float32),
            pltpu.VMEM((1440 * _TW,), jnp.float32),
            pltpu.VMEM((_BPW,), jnp.int32),
            pltpu.VMEM((_BPW,), jnp.int32),
            pltpu.VMEM((_BPW,), jnp.float32),
            pltpu.SemaphoreType.DMA,
            pltpu.SemaphoreType.DMA,
        ],
        compiler_params=pltpu.CompilerParams(
            needs_layout_passes=False, use_tc_tiling_on_sc=True),
    )(_attr_body)
    return run(drv_t, didx, week_tab, widx, time_tab, tidx, dist)


def kernel(driverID, weekID, timeID, dist, W_driverID, W_weekID, W_timeID):
    didx = driverID.astype(jnp.int32)
    widx = weekID.astype(jnp.int32)
    tidx = timeID.astype(jnp.int32)
    dist_f = dist.astype(jnp.float32).reshape(_B)
    week_flat = jnp.pad(W_weekID, ((0, 25), (0, _WPAD - _WW))).reshape(-1)
    time_flat = W_timeID.reshape(-1)
    drv_t = jnp.swapaxes(W_driverID, 0, 1)
    out_t = _attr_call(drv_t, didx, week_flat, widx, time_flat, tidx, dist_f)
    return jnp.swapaxes(out_t, 0, 1)


# speedup vs baseline: 1.0030x; 1.0030x over previous
import functools

import jax
import jax.numpy as jnp
from jax import lax
from jax.experimental import pallas as pl
from jax.experimental.pallas import tpu as pltpu
from jax.experimental.pallas import tpu_sc as plsc

_DIST_MEAN = 9.578281194509781
_DIST_STD = 3.9656010701306283

_B = 16384
_NC = 2
_NS = 16
_NW = _NC * _NS
_BPW = _B // _NW

_DW = 16
_WW = 3
_TW = 8
_OUTW = _DW + _WW + _TW + 1
_WPAD = 4

_WAVE = 16
_NWAVES = _BPW // _WAVE
_BLK = 128


def _attr_body(drv_t, didx, week_tab, widx, time_tab, tidx, dist,
               out_t,
               didx_v, buf0, buf1, panel_v, week_v, time_v, widx_v, tidx_v,
               dist_v, sem0, sem1):
    wid = lax.axis_index("s") * _NC + lax.axis_index("c")
    base = wid * _BPW

    pltpu.sync_copy(didx.at[pl.ds(base, _BPW)], didx_v)
    pltpu.sync_copy(week_tab, week_v)
    pltpu.sync_copy(time_tab, time_v)
    pltpu.sync_copy(widx.at[pl.ds(base, _BPW)], widx_v)
    pltpu.sync_copy(tidx.at[pl.ds(base, _BPW)], tidx_v)
    pltpu.sync_copy(dist.at[pl.ds(base, _BPW)], dist_v)

    lanes = lax.iota(jnp.int32, 16)

    def fire(w, buf, sem):
        jv = didx_v[pl.ds(w * _WAVE, _WAVE)]
        for k in range(_WAVE):
            off = pl.multiple_of(jv[k] & -_BLK, _BLK)
            pltpu.async_copy(drv_t.at[:, pl.ds(off, _BLK)], buf.at[k], sem)
        return jv

    def drain(buf, sem):
        for k in range(_WAVE):
            pltpu.make_async_copy(
                drv_t.at[:, pl.ds(0, _BLK)], buf.at[k], sem).wait()

    def extract(w, jv, buf):
        cols = jv & (_BLK - 1)
        for k in range(_WAVE):
            vals = plsc.load_gather(
                buf, [jnp.full((16,), k, jnp.int32), lanes,
                      jnp.full((16,), cols[k], jnp.int32)])
            plsc.store_scatter(
                panel_v, [lanes, jnp.full((16,), w * _WAVE + k, jnp.int32)],
                vals)

    jv0 = fire(0, buf0, sem0)

    def fire_guarded(w):
        ws = jnp.where(w < _NWAVES, w, _NWAVES - 1)
        return fire(ws, buf0, sem0)

    def wave_pair(t, jv_carry):
        jv_a = jv_carry
        jv_b = fire(2 * t + 1, buf1, sem1)
        drain(buf0, sem0)
        extract(2 * t, jv_a, buf0)
        jv_next = fire_guarded(2 * t + 2)
        drain(buf1, sem1)
        extract(2 * t + 1, jv_b, buf1)
        return jv_next

    lax.fori_loop(0, _NWAVES // 2, wave_pair, jv0)
    drain(buf0, sem0)

    def fill(j, carry):
        r0 = j * 16
        wv = widx_v[pl.ds(r0, 16)]
        tv = tidx_v[pl.ds(r0, 16)]
        dv = dist_v[pl.ds(r0, 16)]
        wb = wv * _WPAD
        tb = tv * _TW
        rvec = r0 + lanes
        for c in range(_WW):
            vals = plsc.load_gather(week_v, [wb + c])
            plsc.store_scatter(
                panel_v, [jnp.full((16,), _DW + c, jnp.int32), rvec], vals)
        for c in range(_TW):
            vals = plsc.load_gather(time_v, [tb + c])
            plsc.store_scatter(
                panel_v, [jnp.full((16,), _DW + _WW + c, jnp.int32), rvec],
                vals)
        d = ((dv - _DIST_MEAN) / _DIST_STD - _DIST_MEAN) / _DIST_STD
        plsc.store_scatter(
            panel_v, [jnp.full((16,), _OUTW - 1, jnp.int32), rvec], d)
        return carry

    lax.fori_loop(0, _BPW // 16, fill, 0)

    pltpu.sync_copy(panel_v, out_t.at[:, pl.ds(base, _BPW)])


@jax.jit
def _attr_call(drv_t, didx, week_tab, widx, time_tab, tidx, dist):
    run = functools.partial(
        pl.kernel,
        out_type=jax.ShapeDtypeStruct((_OUTW, _B), jnp.float32),
        mesh=plsc.VectorSubcoreMesh(core_axis_name="c", subcore_axis_name="s"),
        scratch_types=[
            pltpu.VMEM((_BPW,), jnp.int32),
            pltpu.VMEM((_WAVE, _DW, _BLK), jnp.float32),
            pltpu.VMEM((_WAVE, _DW, _BLK), jnp.float32),
            pltpu.VMEM((_OUTW, _BPW), jnp.float32),
            pltpu.VMEM((32 * _WPAD,), jnp.float32),
            pltpu.VMEM((1440 * _TW,), jnp.float32),
            pltpu.VMEM((_BPW,), jnp.int32),
            pltpu.VMEM((_BPW,), jnp.int32),
            pltpu.VMEM((_BPW,), jnp.float32),
            pltpu.SemaphoreType.DMA,
            pltpu.SemaphoreType.DMA,
        ],
        compiler_params=pltpu.CompilerParams(
            needs_layout_passes=False, use_tc_tiling_on_sc=True),
    )(_attr_body)
    return run(drv_t, didx, week_tab, widx, time_tab, tidx, dist)


def kernel(driverID, weekID, timeID, dist, W_driverID, W_weekID, W_timeID):
    didx = driverID.astype(jnp.int32)
    widx = weekID.astype(jnp.int32)
    tidx = timeID.astype(jnp.int32)
    dist_f = dist.astype(jnp.float32).reshape(_B)
    week_flat = jnp.pad(W_weekID, ((0, 25), (0, _WPAD - _WW))).reshape(-1)
    time_flat = W_timeID.reshape(-1)
    drv_t = jnp.swapaxes(W_driverID, 0, 1)
    out_t = _attr_call(drv_t, didx, week_flat, widx, time_flat, tidx, dist_f)
    return jnp.swapaxes(out_t, 0, 1)


# speedup vs baseline: 1.0231x; 1.0201x over previous
import functools

import jax
import jax.numpy as jnp
from jax import lax
from jax.experimental import pallas as pl
from jax.experimental.pallas import tpu as pltpu
from jax.experimental.pallas import tpu_sc as plsc

_DIST_MEAN = 9.578281194509781
_DIST_STD = 3.9656010701306283

_B = 16384
_NC = 2
_NS = 16
_NW = _NC * _NS
_BPW = _B // _NW

_DW = 16
_WW = 3
_TW = 8
_OUTW = _DW + _WW + _TW + 1
_WPAD = 4

_WAVE = 16
_NWAVES = _BPW // _WAVE
_BLK = 128


def _attr_body(drv_t, didx, week_tab, widx, time_tab, tidx, dist,
               out_t,
               didx_v, buf0, buf1, panel_v, week_v, time_v, widx_v, tidx_v,
               dist_v, sem0, sem1):
    wid = lax.axis_index("s") * _NC + lax.axis_index("c")
    base = wid * _BPW

    pltpu.sync_copy(didx.at[pl.ds(base, _BPW)], didx_v)
    pltpu.sync_copy(week_tab, week_v)
    pltpu.sync_copy(time_tab, time_v)
    pltpu.sync_copy(widx.at[pl.ds(base, _BPW)], widx_v)
    pltpu.sync_copy(tidx.at[pl.ds(base, _BPW)], tidx_v)
    pltpu.sync_copy(dist.at[pl.ds(base, _BPW)], dist_v)

    lanes = lax.iota(jnp.int32, 16)

    def fire(w, buf, sem):
        jv = didx_v[pl.ds(w * _WAVE, _WAVE)]
        for k in range(_WAVE):
            off = pl.multiple_of(jv[k] & -_BLK, _BLK)
            pltpu.async_copy(drv_t.at[:, pl.ds(off, _BLK)], buf.at[k], sem)
        return jv

    def drain(buf, sem):
        for k in range(_WAVE):
            pltpu.make_async_copy(
                drv_t.at[:, pl.ds(0, _BLK)], buf.at[k], sem).wait()

    def extract(w, jv, buf):
        cols = jv & (_BLK - 1)
        rvec = w * _WAVE + lanes
        for r in range(_DW):
            vals = plsc.load_gather(
                buf, [lanes, jnp.full((16,), r, jnp.int32), cols])
            plsc.store_scatter(
                panel_v, [jnp.full((16,), r, jnp.int32), rvec], vals)

    jv0 = fire(0, buf0, sem0)

    def fire_guarded(w):
        ws = jnp.where(w < _NWAVES, w, _NWAVES - 1)
        return fire(ws, buf0, sem0)

    def wave_pair(t, jv_carry):
        jv_a = jv_carry
        jv_b = fire(2 * t + 1, buf1, sem1)
        drain(buf0, sem0)
        extract(2 * t, jv_a, buf0)
        jv_next = fire_guarded(2 * t + 2)
        drain(buf1, sem1)
        extract(2 * t + 1, jv_b, buf1)
        return jv_next

    lax.fori_loop(0, _NWAVES // 2, wave_pair, jv0)
    drain(buf0, sem0)

    def fill(j, carry):
        r0 = j * 16
        wv = widx_v[pl.ds(r0, 16)]
        tv = tidx_v[pl.ds(r0, 16)]
        dv = dist_v[pl.ds(r0, 16)]
        wb = wv * _WPAD
        tb = tv * _TW
        rvec = r0 + lanes
        for c in range(_WW):
            vals = plsc.load_gather(week_v, [wb + c])
            plsc.store_scatter(
                panel_v, [jnp.full((16,), _DW + c, jnp.int32), rvec], vals)
        for c in range(_TW):
            vals = plsc.load_gather(time_v, [tb + c])
            plsc.store_scatter(
                panel_v, [jnp.full((16,), _DW + _WW + c, jnp.int32), rvec],
                vals)
        d = ((dv - _DIST_MEAN) / _DIST_STD - _DIST_MEAN) / _DIST_STD
        plsc.store_scatter(
            panel_v, [jnp.full((16,), _OUTW - 1, jnp.int32), rvec], d)
        return carry

    lax.fori_loop(0, _BPW // 16, fill, 0)

    pltpu.sync_copy(panel_v, out_t.at[:, pl.ds(base, _BPW)])


@jax.jit
def _attr_call(drv_t, didx, week_tab, widx, time_tab, tidx, dist):
    run = functools.partial(
        pl.kernel,
        out_type=jax.ShapeDtypeStruct((_OUTW, _B), jnp.float32),
        mesh=plsc.VectorSubcoreMesh(core_axis_name="c", subcore_axis_name="s"),
        scratch_types=[
            pltpu.VMEM((_BPW,), jnp.int32),
            pltpu.VMEM((_WAVE, _DW, _BLK), jnp.float32),
            pltpu.VMEM((_WAVE, _DW, _BLK), jnp.float32),
            pltpu.VMEM((_OUTW, _BPW), jnp.float32),
            pltpu.VMEM((32 * _WPAD,), jnp.float32),
            pltpu.VMEM((1440 * _TW,), jnp.float32),
            pltpu.VMEM((_BPW,), jnp.int32),
            pltpu.VMEM((_BPW,), jnp.int32),
            pltpu.VMEM((_BPW,), jnp.float32),
            pltpu.SemaphoreType.DMA,
            pltpu.SemaphoreType.DMA,
        ],
        compiler_params=pltpu.CompilerParams(
            needs_layout_passes=False, use_tc_tiling_on_sc=True),
    )(_attr_body)
    return run(drv_t, didx, week_tab, widx, time_tab, tidx, dist)


def kernel(driverID, weekID, timeID, dist, W_driverID, W_weekID, W_timeID):
    didx = driverID.astype(jnp.int32)
    widx = weekID.astype(jnp.int32)
    tidx = timeID.astype(jnp.int32)
    dist_f = dist.astype(jnp.float32).reshape(_B)
    week_flat = jnp.pad(W_weekID, ((0, 25), (0, _WPAD - _WW))).reshape(-1)
    time_flat = W_timeID.reshape(-1)
    drv_t = jnp.swapaxes(W_driverID, 0, 1)
    out_t = _attr_call(drv_t, didx, week_flat, widx, time_flat, tidx, dist_f)
    return jnp.swapaxes(out_t, 0, 1)


# speedup vs baseline: 1.0621x; 1.0381x over previous
import functools

import jax
import jax.numpy as jnp
from jax import lax
from jax.experimental import pallas as pl
from jax.experimental.pallas import tpu as pltpu
from jax.experimental.pallas import tpu_sc as plsc

_DIST_MEAN = 9.578281194509781
_DIST_STD = 3.9656010701306283

_B = 16384
_NC = 2
_NS = 16
_NW = _NC * _NS
_BPW = _B // _NW

_DW = 16
_WW = 3
_TW = 8
_OUTW = _DW + _WW + _TW + 1
_WPAD = 4

_WAVE = 16
_NWAVES = _BPW // _WAVE
_BLK = 128


def _attr_body(drv_t, didx, week_tab, widx, time_tab, tidx, dist,
               out_t,
               didx_v, buf0, buf1, panel_v, week_v, time_v, widx_v, tidx_v,
               dist_v, sem0, sem1, sem2):
    wid = lax.axis_index("s") * _NC + lax.axis_index("c")
    base = wid * _BPW

    pltpu.sync_copy(didx.at[pl.ds(base, _BPW)], didx_v)
    pltpu.async_copy(week_tab, week_v, sem2)
    pltpu.async_copy(time_tab, time_v, sem2)
    pltpu.async_copy(widx.at[pl.ds(base, _BPW)], widx_v, sem2)
    pltpu.async_copy(tidx.at[pl.ds(base, _BPW)], tidx_v, sem2)
    pltpu.async_copy(dist.at[pl.ds(base, _BPW)], dist_v, sem2)

    lanes = lax.iota(jnp.int32, 16)

    def fire(w, buf, sem):
        jv = didx_v[pl.ds(w * _WAVE, _WAVE)]
        for k in range(_WAVE):
            off = pl.multiple_of(jv[k] & -_BLK, _BLK)
            pltpu.async_copy(drv_t.at[:, pl.ds(off, _BLK)], buf.at[k], sem)
        return jv

    def drain(buf, sem):
        for k in range(_WAVE):
            pltpu.make_async_copy(
                drv_t.at[:, pl.ds(0, _BLK)], buf.at[k], sem).wait()

    def extract(w, jv, buf):
        cols = jv & (_BLK - 1)
        rvec = w * _WAVE + lanes
        for r in range(_DW):
            vals = plsc.load_gather(
                buf, [lanes, jnp.full((16,), r, jnp.int32), cols])
            plsc.store_scatter(
                panel_v, [jnp.full((16,), r, jnp.int32), rvec], vals)

    jv0 = fire(0, buf0, sem0)

    def fire_guarded(w):
        ws = jnp.where(w < _NWAVES, w, _NWAVES - 1)
        return fire(ws, buf0, sem0)

    def wave_pair(t, jv_carry):
        jv_a = jv_carry
        jv_b = fire(2 * t + 1, buf1, sem1)
        drain(buf0, sem0)
        extract(2 * t, jv_a, buf0)
        jv_next = fire_guarded(2 * t + 2)
        drain(buf1, sem1)
        extract(2 * t + 1, jv_b, buf1)
        return jv_next

    lax.fori_loop(0, _NWAVES // 2, wave_pair, jv0)
    drain(buf0, sem0)

    pltpu.make_async_copy(week_tab, week_v, sem2).wait()
    pltpu.make_async_copy(time_tab, time_v, sem2).wait()
    pltpu.make_async_copy(widx.at[pl.ds(base, _BPW)], widx_v, sem2).wait()
    pltpu.make_async_copy(tidx.at[pl.ds(base, _BPW)], tidx_v, sem2).wait()
    pltpu.make_async_copy(dist.at[pl.ds(base, _BPW)], dist_v, sem2).wait()

    def fill(j, carry):
        r0 = j * 16
        wv = widx_v[pl.ds(r0, 16)]
        tv = tidx_v[pl.ds(r0, 16)]
        dv = dist_v[pl.ds(r0, 16)]
        wb = wv * _WPAD
        tb = tv * _TW
        rvec = r0 + lanes
        for c in range(_WW):
            vals = plsc.load_gather(week_v, [wb + c])
            plsc.store_scatter(
                panel_v, [jnp.full((16,), _DW + c, jnp.int32), rvec], vals)
        for c in range(_TW):
            vals = plsc.load_gather(time_v, [tb + c])
            plsc.store_scatter(
                panel_v, [jnp.full((16,), _DW + _WW + c, jnp.int32), rvec],
                vals)
        d = ((dv - _DIST_MEAN) / _DIST_STD - _DIST_MEAN) / _DIST_STD
        plsc.store_scatter(
            panel_v, [jnp.full((16,), _OUTW - 1, jnp.int32), rvec], d)
        return carry

    lax.fori_loop(0, _BPW // 16, fill, 0)

    pltpu.sync_copy(panel_v, out_t.at[:, pl.ds(base, _BPW)])


@jax.jit
def _attr_call(drv_t, didx, week_tab, widx, time_tab, tidx, dist):
    run = functools.partial(
        pl.kernel,
        out_type=jax.ShapeDtypeStruct((_OUTW, _B), jnp.float32),
        mesh=plsc.VectorSubcoreMesh(core_axis_name="c", subcore_axis_name="s"),
        scratch_types=[
            pltpu.VMEM((_BPW,), jnp.int32),
            pltpu.VMEM((_WAVE, _DW, _BLK), jnp.float32),
            pltpu.VMEM((_WAVE, _DW, _BLK), jnp.float32),
            pltpu.VMEM((_OUTW, _BPW), jnp.float32),
            pltpu.VMEM((32 * _WPAD,), jnp.float32),
            pltpu.VMEM((1440 * _TW,), jnp.float32),
            pltpu.VMEM((_BPW,), jnp.int32),
            pltpu.VMEM((_BPW,), jnp.int32),
            pltpu.VMEM((_BPW,), jnp.float32),
            pltpu.SemaphoreType.DMA,
            pltpu.SemaphoreType.DMA,
            pltpu.SemaphoreType.DMA,
        ],
        compiler_params=pltpu.CompilerParams(
            needs_layout_passes=False, use_tc_tiling_on_sc=True),
    )(_attr_body)
    return run(drv_t, didx, week_tab, widx, time_tab, tidx, dist)


def kernel(driverID, weekID, timeID, dist, W_driverID, W_weekID, W_timeID):
    didx = driverID.astype(jnp.int32)
    widx = weekID.astype(jnp.int32)
    tidx = timeID.astype(jnp.int32)
    dist_f = dist.astype(jnp.float32).reshape(_B)
    week_flat = jnp.pad(W_weekID, ((0, 25), (0, _WPAD - _WW))).reshape(-1)
    time_flat = W_timeID.reshape(-1)
    drv_t = jnp.swapaxes(W_driverID, 0, 1)
    out_t = _attr_call(drv_t, didx, week_flat, widx, time_flat, tidx, dist_f)
    return jnp.swapaxes(out_t, 0, 1)
